# baseline (speedup 1.0000x reference)
import dataclasses
import functools

import jax
import jax.numpy as jnp
from jax import lax
from jax.experimental import pallas as pl
from jax.experimental.pallas import tpu as pltpu
from jax.experimental.pallas import tpu_sc as plsc

_B, _N, _NF = 32, 6890, 13776
_L = 16
_NP = 6912
_NFP = 13824
_REC_VECS = 3 * _NP // _L
_EDGE_VECS = _NF // _L
_EPS = 1e-5


def _approx_rsqrt(s):
    i = lax.bitcast_convert_type(s, jnp.int32)
    i = jnp.int32(0x5F3759DF) - lax.shift_right_logical(i, 1)
    y = lax.bitcast_convert_type(i, jnp.float32)
    return y * (1.5 - 0.5 * s * y * y)


def _ratio_term(sa, sb):
    r = sa * _approx_rsqrt(sa * sb)
    return jnp.where(sa == sb, 0.0, jnp.abs(r - 1.0))


def _sc_losses(pre_pl, t_pl, f_pl):
    mesh = plsc.VectorSubcoreMesh(core_axis_name="c", subcore_axis_name="s")
    cp = pltpu.CompilerParams()
    if "needs_layout_passes" in pltpu.CompilerParams.__dataclass_fields__:
        cp = dataclasses.replace(cp, needs_layout_passes=False)
    if "use_tc_tiling_on_sc" in pltpu.CompilerParams.__dataclass_fields__:
        cp = dataclasses.replace(cp, use_tc_tiling_on_sc=False)

    @functools.partial(
        pl.kernel,
        compiler_params=cp,
        out_type=[
            jax.ShapeDtypeStruct((_B, _L), jnp.float32),
            jax.ShapeDtypeStruct((_B, _L), jnp.float32),
        ],
        mesh=mesh,
        scratch_types=[
            pltpu.VMEM((3 * _NP,), jnp.float32),
            pltpu.VMEM((3 * _NP,), jnp.float32),
            pltpu.VMEM((3 * _NFP,), jnp.int32),
            pltpu.VMEM((_L,), jnp.float32),
            pltpu.VMEM((_L,), jnp.float32),
            pltpu.SemaphoreType.DMA,
            pltpu.SemaphoreType.DMA,
            pltpu.SemaphoreType.DMA,
            pltpu.SemaphoreType.DMA,
        ],
    )
    def k(pre_hbm, t_hbm, f_hbm, edge_out, rec_out, pv, tv, fv, ea, ra,
          sv0, sv1, sv2, sem_f):
        wid = lax.axis_index("s") * 2 + lax.axis_index("c")
        svs = (sv0, sv1, sv2)
        vcopies = []
        fcopies = []
        for c in range(3):
            vcopies.append((
                pltpu.async_copy(
                    pre_hbm.at[c, wid], pv.at[pl.ds(c * _NP, _NP)], svs[c]),
                pltpu.async_copy(
                    t_hbm.at[c, wid], tv.at[pl.ds(c * _NP, _NP)], svs[c]),
            ))
            fcopies.append(pltpu.async_copy(
                f_hbm.at[c, wid], fv.at[pl.ds(c * _NFP, _NFP)], sem_f))

        _PL_VECS = _NP // _L

        def rec_one(base, j, acc):
            d = pv[pl.ds(base + j * _L, _L)] - tv[pl.ds(base + j * _L, _L)]
            return acc + d * d

        rec = jnp.zeros((_L,), jnp.float32)
        for c in range(3):
            vcopies[c][0].wait()
            vcopies[c][1].wait()

            def rec_body(i, acc, base=c * _NP):
                return rec_one(base, 2 * i + 1, rec_one(base, 2 * i, acc))

            rec = lax.fori_loop(0, _PL_VECS // 2, rec_body, rec)

        for cp_ in fcopies:
            cp_.wait()

        planes = tuple(
            tab.at[pl.ds(c * _NP, _NP)] for tab in (pv, tv) for c in range(3))

        def pt(which, ix):
            return (plsc.load_gather(planes[3 * which + 0], [ix]),
                    plsc.load_gather(planes[3 * which + 1], [ix]),
                    plsc.load_gather(planes[3 * which + 2], [ix]))

        def d2(a, b):
            dx = a[0] - b[0]
            dy = a[1] - b[1]
            dz = a[2] - b[2]
            return dx * dx + dy * dy + dz * dz

        def edge_one(j, acc):
            f0 = fv[pl.ds(j * _L, _L)]
            f1 = fv[pl.ds(_NFP + j * _L, _L)]
            f2 = fv[pl.ds(2 * _NFP + j * _L, _L)]
            ap1, ap2, ap3 = pt(0, f0), pt(0, f1), pt(0, f2)
            bp1, bp2, bp3 = pt(1, f0), pt(1, f1), pt(1, f2)
            term = (_ratio_term(d2(ap1, ap2), d2(bp1, bp2))
                    + _ratio_term(d2(ap1, ap3), d2(bp1, bp3))
                    + _ratio_term(d2(ap2, ap3), d2(bp2, bp3)))
            return acc + term

        def edge_body(i, acc):
            for u in range(4):
                acc = edge_one(4 * i + u, acc)
            return acc

        edge = lax.fori_loop(0, _EDGE_VECS // 4, edge_body,
                             jnp.zeros((_L,), jnp.float32))
        edge = edge_one(_EDGE_VECS - 1, edge)

        ea[...] = edge
        ra[...] = rec
        pltpu.sync_copy(ea, edge_out.at[wid])
        pltpu.sync_copy(ra, rec_out.at[wid])

    return k(pre_pl, t_pl, f_pl)


def kernel(preV, tV, F):
    pre_pl = jnp.pad(jnp.transpose(preV, (2, 0, 1)),
                     ((0, 0), (0, 0), (0, _NP - _N)))
    t_pl = jnp.pad(jnp.transpose(tV, (2, 0, 1)),
                   ((0, 0), (0, 0), (0, _NP - _N)))
    f_pl = jnp.pad(jnp.transpose(F, (2, 0, 1)),
                   ((0, 0), (0, 0), (0, _NFP - _NF)))
    edge_p, rec_p = _sc_losses(pre_pl, t_pl, f_pl)
    rec_loss = jnp.sum(rec_p) / jnp.float32(_B * _N * 3)
    edg_loss = jnp.sum(edge_p) / jnp.float32(_B * _NF)
    return rec_loss, edg_loss


# speedup vs baseline: 1.0295x; 1.0295x over previous
import dataclasses
import functools

import jax
import jax.numpy as jnp
from jax import lax
from jax.experimental import pallas as pl
from jax.experimental.pallas import tpu as pltpu
from jax.experimental.pallas import tpu_sc as plsc

_B, _N, _NF = 32, 6890, 13776
_L = 16
_NP = 6912
_NFP = 13824
_REC_VECS = 3 * _NP // _L
_EDGE_VECS = _NF // _L
_EPS = 1e-5


def _approx_rsqrt(s):
    i = lax.bitcast_convert_type(s, jnp.int32)
    i = jnp.int32(0x5F3759DF) - lax.shift_right_logical(i, 1)
    y = lax.bitcast_convert_type(i, jnp.float32)
    return y * (1.5 - 0.5 * s * y * y)


def _ratio_term(sa, sb):
    r = sa * _approx_rsqrt(sa * sb)
    return jnp.where(sa == sb, 0.0, jnp.abs(r - 1.0))


def _sc_losses(pre_pl, t_pl, f_pl):
    mesh = plsc.VectorSubcoreMesh(core_axis_name="c", subcore_axis_name="s")
    cp = pltpu.CompilerParams()
    if "needs_layout_passes" in pltpu.CompilerParams.__dataclass_fields__:
        cp = dataclasses.replace(cp, needs_layout_passes=False)
    if "use_tc_tiling_on_sc" in pltpu.CompilerParams.__dataclass_fields__:
        cp = dataclasses.replace(cp, use_tc_tiling_on_sc=False)

    @functools.partial(
        pl.kernel,
        compiler_params=cp,
        out_type=[
            jax.ShapeDtypeStruct((_B, _L), jnp.float32),
            jax.ShapeDtypeStruct((_B, _L), jnp.float32),
        ],
        mesh=mesh,
        scratch_types=[
            pltpu.VMEM((3 * _NP,), jnp.float32),
            pltpu.VMEM((3 * _NP,), jnp.float32),
            pltpu.VMEM((3 * _NFP,), jnp.int32),
            pltpu.VMEM((_L,), jnp.float32),
            pltpu.VMEM((_L,), jnp.float32),
            pltpu.SemaphoreType.DMA,
            pltpu.SemaphoreType.DMA,
            pltpu.SemaphoreType.DMA,
            pltpu.SemaphoreType.DMA,
        ],
    )
    def k(pre_hbm, t_hbm, f_hbm, edge_out, rec_out, pv, tv, fv, ea, ra,
          sv0, sv1, sv2, sem_f):
        wid = lax.axis_index("s") * 2 + lax.axis_index("c")
        svs = (sv0, sv1, sv2)
        vcopies = []
        fcopies = []
        for c in range(3):
            vcopies.append((
                pltpu.async_copy(
                    pre_hbm.at[c, wid], pv.at[pl.ds(c * _NP, _NP)], svs[c]),
                pltpu.async_copy(
                    t_hbm.at[c, wid], tv.at[pl.ds(c * _NP, _NP)], svs[c]),
            ))
            fcopies.append(pltpu.async_copy(
                f_hbm.at[c, wid], fv.at[pl.ds(c * _NFP, _NFP)], sem_f))

        _PL_VECS = _NP // _L

        def rec_one(base, j, acc):
            d = pv[pl.ds(base + j * _L, _L)] - tv[pl.ds(base + j * _L, _L)]
            return acc + d * d

        rec = jnp.zeros((_L,), jnp.float32)
        for c in range(3):
            vcopies[c][0].wait()
            vcopies[c][1].wait()

            def rec_body(i, acc, base=c * _NP):
                return rec_one(base, 2 * i + 1, rec_one(base, 2 * i, acc))

            rec = lax.fori_loop(0, _PL_VECS // 2, rec_body, rec)

        for cp_ in fcopies:
            cp_.wait()

        planes = tuple(
            tab.at[pl.ds(c * _NP, _NP)] for tab in (pv, tv) for c in range(3))

        def pt(which, ix):
            return (plsc.load_gather(planes[3 * which + 0], [ix]),
                    plsc.load_gather(planes[3 * which + 1], [ix]),
                    plsc.load_gather(planes[3 * which + 2], [ix]))

        def d2(a, b):
            dx = a[0] - b[0]
            dy = a[1] - b[1]
            dz = a[2] - b[2]
            return dx * dx + dy * dy + dz * dz

        def edge_one(j, acc):
            f0 = fv[pl.ds(j * _L, _L)]
            f1 = fv[pl.ds(_NFP + j * _L, _L)]
            f2 = fv[pl.ds(2 * _NFP + j * _L, _L)]
            ap1, ap2, ap3 = pt(0, f0), pt(0, f1), pt(0, f2)
            bp1, bp2, bp3 = pt(1, f0), pt(1, f1), pt(1, f2)
            term = (_ratio_term(d2(ap1, ap2), d2(bp1, bp2))
                    + _ratio_term(d2(ap1, ap3), d2(bp1, bp3))
                    + _ratio_term(d2(ap2, ap3), d2(bp2, bp3)))
            return acc + term

        def edge_body(i, acc):
            return edge_one(3 * i + 2, edge_one(3 * i + 1, edge_one(3 * i, acc)))

        edge = lax.fori_loop(0, _EDGE_VECS // 3, edge_body,
                             jnp.zeros((_L,), jnp.float32))

        ea[...] = edge
        ra[...] = rec
        pltpu.sync_copy(ea, edge_out.at[wid])
        pltpu.sync_copy(ra, rec_out.at[wid])

    return k(pre_pl, t_pl, f_pl)


def kernel(preV, tV, F):
    pre_pl = jnp.pad(jnp.transpose(preV, (2, 0, 1)),
                     ((0, 0), (0, 0), (0, _NP - _N)))
    t_pl = jnp.pad(jnp.transpose(tV, (2, 0, 1)),
                   ((0, 0), (0, 0), (0, _NP - _N)))
    f_pl = jnp.pad(jnp.transpose(F, (2, 0, 1)),
                   ((0, 0), (0, 0), (0, _NFP - _NF)))
    edge_p, rec_p = _sc_losses(pre_pl, t_pl, f_pl)
    rec_loss = jnp.sum(rec_p) / jnp.float32(_B * _N * 3)
    edg_loss = jnp.sum(edge_p) / jnp.float32(_B * _NF)
    return rec_loss, edg_loss


# speedup vs baseline: 1.0300x; 1.0005x over previous
import dataclasses
import functools

import jax
import jax.numpy as jnp
from jax import lax
from jax.experimental import pallas as pl
from jax.experimental.pallas import tpu as pltpu
from jax.experimental.pallas import tpu_sc as plsc

_B, _N, _NF = 32, 6890, 13776
_L = 16
_NP = 6912
_NFP = 13824
_EDGE_VECS = _NF // _L


def _approx_rsqrt(s):
    i = lax.bitcast_convert_type(s, jnp.int32)
    i = jnp.int32(0x5F3759DF) - lax.shift_right_logical(i, 1)
    y = lax.bitcast_convert_type(i, jnp.float32)
    return y * (1.5 - 0.5 * s * y * y)


def _ratio_term(sa, sb):
    r = sa * _approx_rsqrt(sa * sb)
    return jnp.where(sa == sb, 0.0, jnp.abs(r - 1.0))


def _sc_losses(pre_pl, t_pl, f_pl):
    mesh = plsc.VectorSubcoreMesh(core_axis_name="c", subcore_axis_name="s")
    cp = pltpu.CompilerParams()
    if "needs_layout_passes" in pltpu.CompilerParams.__dataclass_fields__:
        cp = dataclasses.replace(cp, needs_layout_passes=False)
    if "use_tc_tiling_on_sc" in pltpu.CompilerParams.__dataclass_fields__:
        cp = dataclasses.replace(cp, use_tc_tiling_on_sc=False)

    @functools.partial(
        pl.kernel,
        compiler_params=cp,
        out_type=[
            jax.ShapeDtypeStruct((_B, _L), jnp.float32),
            jax.ShapeDtypeStruct((_B, _L), jnp.float32),
        ],
        mesh=mesh,
        scratch_types=[
            pltpu.VMEM((3 * _NP,), jnp.float32),
            pltpu.VMEM((3 * _NP,), jnp.float32),
            pltpu.VMEM((3 * _NFP,), jnp.int32),
            pltpu.VMEM((_L,), jnp.float32),
            pltpu.VMEM((_L,), jnp.float32),
            pltpu.SemaphoreType.DMA,
            pltpu.SemaphoreType.DMA,
            pltpu.SemaphoreType.DMA,
            pltpu.SemaphoreType.DMA,
        ],
    )
    def k(pre_hbm, t_hbm, f_hbm, edge_out, rec_out, pv, tv, fv, ea, ra,
          sv0, sv1, sv2, sem_f):
        wid = lax.axis_index("s") * 2 + lax.axis_index("c")
        svs = (sv0, sv1, sv2)
        vcopies = []
        fcopies = []
        for c in range(3):
            vcopies.append((
                pltpu.async_copy(
                    pre_hbm.at[c, wid], pv.at[pl.ds(c * _NP, _NP)], svs[c]),
                pltpu.async_copy(
                    t_hbm.at[c, wid], tv.at[pl.ds(c * _NP, _NP)], svs[c]),
            ))
            fcopies.append(pltpu.async_copy(
                f_hbm.at[c, wid], fv.at[pl.ds(c * _NFP, _NFP)], sem_f))

        _PL_VECS = _NP // _L

        def rec_one(base, j, acc):
            d = pv[pl.ds(base + j * _L, _L)] - tv[pl.ds(base + j * _L, _L)]
            return acc + d * d

        rec = jnp.zeros((_L,), jnp.float32)
        for c in range(3):
            vcopies[c][0].wait()
            vcopies[c][1].wait()

            def rec_body(i, acc, base=c * _NP):
                return rec_one(base, 2 * i + 1, rec_one(base, 2 * i, acc))

            rec = lax.fori_loop(0, _PL_VECS // 2, rec_body, rec)

        for cp_ in fcopies:
            cp_.wait()

        planes = tuple(
            tab.at[pl.ds(c * _NP, _NP)] for tab in (pv, tv) for c in range(3))

        def pt(which, ix):
            return (plsc.load_gather(planes[3 * which + 0], [ix]),
                    plsc.load_gather(planes[3 * which + 1], [ix]),
                    plsc.load_gather(planes[3 * which + 2], [ix]))

        def d2(a, b):
            dx = a[0] - b[0]
            dy = a[1] - b[1]
            dz = a[2] - b[2]
            return dx * dx + dy * dy + dz * dz

        def edge_one(j, acc):
            f0 = fv[pl.ds(j * _L, _L)]
            f1 = fv[pl.ds(_NFP + j * _L, _L)]
            f2 = fv[pl.ds(2 * _NFP + j * _L, _L)]
            ap1, ap2, ap3 = pt(0, f0), pt(0, f1), pt(0, f2)
            bp1, bp2, bp3 = pt(1, f0), pt(1, f1), pt(1, f2)
            term = (_ratio_term(d2(ap1, ap2), d2(bp1, bp2))
                    + _ratio_term(d2(ap1, ap3), d2(bp1, bp3))
                    + _ratio_term(d2(ap2, ap3), d2(bp2, bp3)))
            return acc + term

        def edge_body(i, acc):
            return edge_one(3 * i + 2, edge_one(3 * i + 1, edge_one(3 * i, acc)))

        edge = lax.fori_loop(0, _EDGE_VECS // 3, edge_body,
                             jnp.zeros((_L,), jnp.float32))

        ea[...] = edge
        ra[...] = rec
        pltpu.sync_copy(ea, edge_out.at[wid])
        pltpu.sync_copy(ra, rec_out.at[wid])

    return k(pre_pl, t_pl, f_pl)


def kernel(preV, tV, F):
    pre_pl = jnp.pad(jnp.transpose(preV, (2, 0, 1)),
                     ((0, 0), (0, 0), (0, _NP - _N)))
    t_pl = jnp.pad(jnp.transpose(tV, (2, 0, 1)),
                   ((0, 0), (0, 0), (0, _NP - _N)))
    f_pl = jnp.pad(jnp.transpose(F, (2, 0, 1)),
                   ((0, 0), (0, 0), (0, _NFP - _NF)))
    edge_p, rec_p = _sc_losses(pre_pl, t_pl, f_pl)
    rec_loss = jnp.sum(rec_p) / jnp.float32(_B * _N * 3)
    edg_loss = jnp.sum(edge_p) / jnp.float32(_B * _NF)
    return rec_loss, edg_loss
